# baseline (speedup 1.0000x reference)
import jax
import jax.numpy as jnp
from jax.experimental import pallas as pl
from jax.experimental.pallas import tpu as pltpu
from jax.experimental.pallas import tpu_sc as plsc

BATCH = 16384
HIST = 200
EMB = 32
N = BATCH * HIST
WINDOW = 128
K = 8


def _sc_gather(table_hbm, idx_hbm, out_hbm, sem):
    def body(i_vmem, o_vmem):
        copies = [
            pltpu.async_copy(
                table_hbm.at[i_vmem.at[j]],
                o_vmem.at[pl.ds(j * WINDOW, WINDOW)],
                sem,
            )
            for j in range(K)
        ]
        for c in copies:
            c.wait()

    pltpu.emit_pipeline(
        body,
        grid=(N // (WINDOW * K),),
        in_specs=[pl.BlockSpec((K, WINDOW), index_map=lambda i: (i, 0))],
        out_specs=[pl.BlockSpec((K * WINDOW, EMB), index_map=lambda i: (i, 0))],
        core_axis_name=("c", "s"),
        dimension_semantics=(pltpu.PARALLEL,),
    )(idx_hbm, out_hbm)


@jax.jit
def kernel(prev_action, action_emb_weight):
    idx = prev_action.reshape(N // WINDOW, WINDOW).astype(jnp.int32)
    mesh = plsc.VectorSubcoreMesh(core_axis_name="c", subcore_axis_name="s")
    out = pl.kernel(
        _sc_gather,
        out_type=jax.ShapeDtypeStruct((N, EMB), jnp.float32),
        mesh=mesh,
        scratch_types=[pltpu.SemaphoreType.DMA],
        compiler_params=pltpu.CompilerParams(use_tc_tiling_on_sc=False),
    )(action_emb_weight, idx)
    return out.reshape(BATCH, HIST, EMB)
